# Initial kernel scaffold; baseline (speedup 1.0000x reference)
import jax
import jax.numpy as jnp
from jax.experimental import pallas as pl


def kernel(x, edge_index, edge_weight, Wc, bc, W1, b1, W2, b2):
    raise NotImplementedError("write your pallas kernel here")


# speedup vs baseline: 66.1265x; 66.1265x over previous
import functools

import jax
import jax.numpy as jnp
from jax import lax
from jax.experimental import pallas as pl
from jax.experimental.pallas import tpu as pltpu
from jax.experimental.pallas import tpu_sc as plsc

N_NODES = 10000
HID = 128
PRE_LEN = 12
LANES = 16
N_PAD = 10240
NC = 2
NS = 16
NW = NC * NS
CHUNK = 128
EPW_CHUNKS = 79
E_PAD = NW * EPW_CHUNKS * CHUNK
ROWS_PER_TILE = N_PAD // NS

NB = 512
N_BLOCKS = N_PAD // NB


def _deg_body(col_hbm, ewb_hbm, zeros_hbm, deg_out, col_v, buf_v, deg_sh):
    c = lax.axis_index("c")
    s = lax.axis_index("s")
    wid = c * NS + s
    pltpu.sync_copy(col_hbm.at[wid], col_v)
    pltpu.sync_copy(zeros_hbm.at[pl.ds(s * ROWS_PER_TILE, ROWS_PER_TILE)],
                    deg_sh.at[pl.ds(s * ROWS_PER_TILE, ROWS_PER_TILE)])
    plsc.subcore_barrier()

    def chunk(j, carry):
        pltpu.sync_copy(ewb_hbm.at[wid, j], buf_v)
        pltpu.sync_copy(buf_v, deg_sh.at[col_v.at[j]], add=True)
        return carry

    lax.fori_loop(0, EPW_CHUNKS, chunk, 0)
    plsc.subcore_barrier()
    pltpu.sync_copy(deg_sh.at[pl.ds(s * ROWS_PER_TILE, ROWS_PER_TILE)],
                    deg_out.at[c, pl.ds(s * ROWS_PER_TILE, ROWS_PER_TILE)])


@functools.cache
def _make_deg_kernel():
    return pl.kernel(
        _deg_body,
        out_type=jax.ShapeDtypeStruct((NC, N_PAD, LANES), jnp.float32),
        mesh=plsc.VectorSubcoreMesh(core_axis_name="c", subcore_axis_name="s"),
        scratch_types=[
            pltpu.VMEM((EPW_CHUNKS, CHUNK), jnp.int32),
            pltpu.VMEM((CHUNK, LANES), jnp.float32),
            pltpu.VMEM_SHARED((N_PAD, LANES), jnp.float32),
        ],
        compiler_params=pltpu.CompilerParams(use_tc_tiling_on_sc=False),
    )


def _prep_body(degmat_ref, x_ref, y_ref, dinv_ref):
    deg = degmat_ref[0] + degmat_ref[1] + 2.0
    dinv = lax.rsqrt(deg)
    dinv_ref[...] = dinv
    y_ref[...] = x_ref[...] * dinv


def _run_prep(degmat, x_r):
    R = N_PAD * LANES // 128
    return pl.pallas_call(
        _prep_body,
        out_shape=[jax.ShapeDtypeStruct((R, 128), jnp.float32),
                   jax.ShapeDtypeStruct((R, 128), jnp.float32)],
    )(degmat, x_r)


def _z_body(row_hbm, col_hbm, ewb_hbm, y_hbm, zeros_hbm, z_out,
            row_v, col_v, ewbuf, ybuf, zbuf, z_sh, sem):
    c = lax.axis_index("c")
    s = lax.axis_index("s")
    wid = c * NS + s
    pltpu.sync_copy(row_hbm.at[wid], row_v)
    pltpu.sync_copy(col_hbm.at[wid], col_v)
    pltpu.sync_copy(zeros_hbm.at[pl.ds(s * ROWS_PER_TILE, ROWS_PER_TILE)],
                    z_sh.at[pl.ds(s * ROWS_PER_TILE, ROWS_PER_TILE)])
    plsc.subcore_barrier()

    def chunk(j, carry):
        copy = pltpu.async_copy(y_hbm.at[row_v.at[j]], ybuf, sem)
        pltpu.sync_copy(ewb_hbm.at[wid, j], ewbuf)
        copy.wait()
        for r in range(CHUNK):
            zbuf[r, :] = ybuf[r, :] * ewbuf[r, :]
        pltpu.sync_copy(zbuf, z_sh.at[col_v.at[j]], add=True)
        return carry

    lax.fori_loop(0, EPW_CHUNKS, chunk, 0)
    plsc.subcore_barrier()
    pltpu.sync_copy(z_sh.at[pl.ds(s * ROWS_PER_TILE, ROWS_PER_TILE)],
                    z_out.at[c, pl.ds(s * ROWS_PER_TILE, ROWS_PER_TILE)])


@functools.cache
def _make_z_kernel():
    return pl.kernel(
        _z_body,
        out_type=jax.ShapeDtypeStruct((NC, N_PAD, LANES), jnp.float32),
        mesh=plsc.VectorSubcoreMesh(core_axis_name="c", subcore_axis_name="s"),
        scratch_types=[
            pltpu.VMEM((EPW_CHUNKS, CHUNK), jnp.int32),
            pltpu.VMEM((EPW_CHUNKS, CHUNK), jnp.int32),
            pltpu.VMEM((CHUNK, LANES), jnp.float32),
            pltpu.VMEM((CHUNK, LANES), jnp.float32),
            pltpu.VMEM((CHUNK, LANES), jnp.float32),
            pltpu.VMEM_SHARED((N_PAD, LANES), jnp.float32),
            pltpu.SemaphoreType.DMA,
        ],
        compiler_params=pltpu.CompilerParams(use_tc_tiling_on_sc=False),
    )


def _gru_body(z_ref, y_ref, dinv_ref, wc_ref, bc_ref, w1_ref, b1_ref,
              w2_ref, b2_ref, out_ref):
    dinv = dinv_ref[...]
    s_all = dinv * (z_ref[0] + z_ref[1] + 2.0 * y_ref[...])
    wc = wc_ref[...]
    bc = bc_ref[...]
    b1 = b1_ref[...]
    b2 = b2_ref[...]
    w1 = w1_ref[...]
    w2 = w2_ref[...]
    h = jnp.zeros((NB, HID), jnp.float32)
    for t in range(PRE_LEN):
        st = s_all[:, t:t + 1]
        f = jax.nn.sigmoid(st * wc + bc)
        cat1 = jnp.concatenate([f, h], axis=1)
        ru = jax.nn.sigmoid(
            jnp.dot(cat1, w1, preferred_element_type=jnp.float32) + b1)
        r = ru[:, :HID]
        u = ru[:, HID:]
        cat2 = jnp.concatenate([f, r * h], axis=1)
        cnew = jnp.tanh(
            jnp.dot(cat2, w2, preferred_element_type=jnp.float32) + b2)
        h = u * h + (1.0 - u) * cnew
    out_ref[...] = h


def _run_gru(zmat, y2, dinv2, Wc, bc, W1, b1, W2, b2):
    grid = (N_BLOCKS,)
    return pl.pallas_call(
        _gru_body,
        grid=grid,
        in_specs=[
            pl.BlockSpec((NC, NB, LANES), lambda i: (0, i, 0)),
            pl.BlockSpec((NB, LANES), lambda i: (i, 0)),
            pl.BlockSpec((NB, LANES), lambda i: (i, 0)),
            pl.BlockSpec((1, HID), lambda i: (0, 0)),
            pl.BlockSpec((1, HID), lambda i: (0, 0)),
            pl.BlockSpec((2 * HID, 2 * HID), lambda i: (0, 0)),
            pl.BlockSpec((1, 2 * HID), lambda i: (0, 0)),
            pl.BlockSpec((2 * HID, HID), lambda i: (0, 0)),
            pl.BlockSpec((1, HID), lambda i: (0, 0)),
        ],
        out_specs=pl.BlockSpec((NB, HID), lambda i: (i, 0)),
        out_shape=jax.ShapeDtypeStruct((N_PAD, HID), jnp.float32),
    )(zmat, y2, dinv2, Wc, bc, W1, b1, W2, b2)


def kernel(x, edge_index, edge_weight, Wc, bc, W1, b1, W2, b2):
    E = edge_weight.shape[0]
    row = edge_index[0].astype(jnp.int32)
    col = edge_index[1].astype(jnp.int32)
    ew = edge_weight.astype(jnp.float32)

    pad_e = E_PAD - E
    row3 = jnp.pad(row, (0, pad_e)).reshape(NW, EPW_CHUNKS, CHUNK)
    col3 = jnp.pad(col, (0, pad_e)).reshape(NW, EPW_CHUNKS, CHUNK)
    ewp = jnp.pad(ew, (0, pad_e))
    ewb = jnp.broadcast_to(ewp[:, None], (E_PAD, LANES)).reshape(
        NW, EPW_CHUNKS, CHUNK, LANES)

    x_pad = jnp.pad(x, ((0, N_PAD - N_NODES), (0, LANES - PRE_LEN)))
    zeros_pad = jnp.zeros((N_PAD, LANES), jnp.float32)

    degmat = _make_deg_kernel()(col3, ewb, zeros_pad)

    R = N_PAD * LANES // 128
    y_r, dinv_r = _run_prep(degmat.reshape(NC, R, 128), x_pad.reshape(R, 128))
    y2 = y_r.reshape(N_PAD, LANES)
    dinv2 = dinv_r.reshape(N_PAD, LANES)

    zmat = _make_z_kernel()(row3, col3, ewb, y2, zeros_pad)

    h = _run_gru(zmat, y2, dinv2, Wc, bc.reshape(1, HID), W1,
                 b1.reshape(1, 2 * HID), W2, b2.reshape(1, HID))
    return h[:N_NODES]
